# baseline (speedup 1.0000x reference)
import functools

import jax
import jax.numpy as jnp
from jax import lax
from jax.experimental import pallas as pl
from jax.experimental.pallas import tpu as pltpu
from jax.experimental.pallas import tpu_sc as plsc

N = 50000
D = 66
DP = 72
OUT = 22
EPS = 1e-5

NC = 2
NS = 16
NW = NC * NS

E = 800000
CH = 64
SB = 1024
SB2 = 1024
NSB = 50
NSB2 = 25
EPT = NSB * SB
EPT2 = NSB2 * SB2
E_PAD = 16 * EPT

NHALF = N // 2
RPT = 1563
TRASH0 = 16 * RPT
SP_ROWS = TRASH0 + 256
BLK = 5000

_mesh = plsc.VectorSubcoreMesh(core_axis_name="c", subcore_axis_name="s")
_sc_params = pltpu.CompilerParams(
    needs_layout_passes=False, use_tc_tiling_on_sc=False)


def _sc_deg_body(col_hbm, ew_hbm, degp_hbm, colv, ewv, degv):
    cid = lax.axis_index("c")
    sid = lax.axis_index("s")
    wid = cid * NS + sid

    def _zero(i, _):
        degv[pl.ds(i * 16, 16)] = jnp.zeros((16,), jnp.float32)
        return 0

    lax.fori_loop(0, N // 16, _zero, 0)

    def _sb(s, _):
        eoff = wid * EPT2 + s * SB2
        pltpu.sync_copy(col_hbm.at[pl.ds(eoff, SB2)], colv)
        pltpu.sync_copy(ew_hbm.at[pl.ds(eoff, SB2)], ewv)

        def _q(q, _):
            c16 = colv[pl.ds(q * 16, 16)]
            w16 = ewv[pl.ds(q * 16, 16)]
            plsc.addupdate_scatter(degv, [c16], w16)
            return 0

        lax.fori_loop(0, SB2 // 16, _q, 0)
        return 0

    lax.fori_loop(0, NSB2, _sb, 0)
    pltpu.sync_copy(degv, degp_hbm.at[wid])


def _sc_norm_body(dis_hbm, row_hbm, col_hbm, ew_hbm, norm_hbm,
                  disv, rowv, colv, ewv, noutv):
    cid = lax.axis_index("c")
    sid = lax.axis_index("s")
    wid = cid * NS + sid
    pltpu.sync_copy(dis_hbm, disv)

    def _sb(s, _):
        eoff = wid * EPT2 + s * SB2
        pltpu.sync_copy(row_hbm.at[pl.ds(eoff, SB2)], rowv)
        pltpu.sync_copy(col_hbm.at[pl.ds(eoff, SB2)], colv)
        pltpu.sync_copy(ew_hbm.at[pl.ds(eoff, SB2)], ewv)

        def _q(q, _):
            r16 = rowv[pl.ds(q * 16, 16)]
            c16 = colv[pl.ds(q * 16, 16)]
            w16 = ewv[pl.ds(q * 16, 16)]
            dr = plsc.load_gather(disv, [r16])
            dc = plsc.load_gather(disv, [c16])
            noutv[pl.ds(q * 16, 16)] = dr * w16 * dc
            return 0

        lax.fori_loop(0, SB2 // 16, _q, 0)
        pltpu.sync_copy(noutv, norm_hbm.at[pl.ds(eoff, SB2)])
        return 0

    lax.fori_loop(0, NSB2, _sb, 0)


def _sc_agg_body(y_hbm, row_hbm, col_hbm, norm_hbm, agg_hbm,
                 rowi, coli, nrm, gath2, sgath, cidx, shared, sem):
    cid = lax.axis_index("c")
    sid = lax.axis_index("s")
    base = cid * NHALF
    NCH = SB // CH

    def _zb(i, _):
        for f in range(4):
            gath2[0, i, pl.ds(f * 16, 16)] = jnp.zeros((16,), jnp.float32)
        gath2[0, i, pl.ds(DP - 16, 16)] = jnp.zeros((16,), jnp.float32)
        return 0

    lax.fori_loop(0, CH, _zb, 0)

    def _zs(j, _):
        pltpu.sync_copy(gath2.at[0],
                        shared.at[pl.ds(sid * RPT + j * CH, CH)])
        return 0

    lax.fori_loop(0, RPT // CH, _zs, 0)
    pltpu.sync_copy(gath2.at[0, pl.ds(0, RPT % CH)],
                    shared.at[pl.ds(sid * RPT + RPT - RPT % CH, RPT % CH)])
    pltpu.sync_copy(gath2.at[0, pl.ds(0, 16)],
                    shared.at[pl.ds(TRASH0 + sid * 16, 16)])
    plsc.subcore_barrier()

    def _sb(s, _):
        eoff = sid * EPT + s * SB
        pltpu.sync_copy(row_hbm.at[pl.ds(eoff, SB)], rowi)
        pltpu.sync_copy(col_hbm.at[pl.ds(eoff, SB)], coli)
        pltpu.sync_copy(norm_hbm.at[pl.ds(eoff, SB)], nrm)
        pltpu.async_copy(y_hbm.at[rowi.at[pl.ds(0, CH)]], gath2.at[0], sem)

        def _pair(cp, _):
            for p in range(2):
                ch = cp * 2 + p
                co = ch * CH
                gbuf = gath2.at[p]
                pltpu.make_async_copy(
                    y_hbm.at[rowi.at[pl.ds(0, CH)]], gbuf, sem).wait()

                @pl.when(ch < NCH - 1)
                def _issue():
                    pltpu.async_copy(
                        y_hbm.at[rowi.at[pl.ds(co + CH, CH)]],
                        gath2.at[1 - p], sem)

                def _q(q, _):
                    cv = coli[pl.ds(co + q * 16, 16)] - base
                    okm = (cv >= 0) & (cv < NHALF)
                    tv = TRASH0 + sid * 16 + lax.iota(jnp.int32, 16)
                    cidx[pl.ds(q * 16, 16)] = jnp.where(okm, cv, tv)
                    wv = nrm[pl.ds(co + q * 16, 16)]
                    e0 = q * 16
                    for i in range(16):
                        w = wv[i]

                        @pl.when(okm.astype(jnp.int32)[i] != 0)
                        def _scale_row():
                            for f in range(4):
                                sgath[e0 + i, pl.ds(f * 16, 16)] = (
                                    gbuf[e0 + i, pl.ds(f * 16, 16)] * w)
                            sgath[e0 + i, pl.ds(DP - 16, 16)] = (
                                gbuf[e0 + i, pl.ds(DP - 16, 16)] * w)
                    return 0

                lax.fori_loop(0, CH // 16, _q, 0)
                pltpu.sync_copy(sgath, shared.at[cidx], add=True)
            return 0

        lax.fori_loop(0, NCH // 2, _pair, 0)
        return 0

    lax.fori_loop(0, NSB, _sb, 0)
    plsc.subcore_barrier()

    def _out(j, _):
        off = sid * RPT + j * CH
        pltpu.sync_copy(shared.at[pl.ds(off, CH)],
                        agg_hbm.at[pl.ds(cid * NHALF + off, CH)])
        return 0

    lax.fori_loop(0, RPT // CH, _out, 0)
    off2 = sid * RPT + RPT - RPT % CH

    @pl.when(sid < NS - 1)
    def _tail_full():
        pltpu.sync_copy(shared.at[pl.ds(off2, RPT % CH)],
                        agg_hbm.at[pl.ds(cid * NHALF + off2, RPT % CH)])

    @pl.when(sid == NS - 1)
    def _tail_last():
        rem = NHALF - (NS - 1) * RPT - (RPT // CH) * CH
        pltpu.sync_copy(shared.at[pl.ds(off2, rem)],
                        agg_hbm.at[pl.ds(cid * NHALF + off2, rem)])


def _make_sc_kernels(interpret=False):
    deg = pl.kernel(
        _sc_deg_body,
        out_type=jax.ShapeDtypeStruct((NW, N), jnp.float32),
        mesh=_mesh,
        compiler_params=_sc_params,
        interpret=interpret,
        scratch_types=[
            pltpu.VMEM((SB2,), jnp.int32),
            pltpu.VMEM((SB2,), jnp.float32),
            pltpu.VMEM((N,), jnp.float32),
        ],
    )
    nrm = pl.kernel(
        _sc_norm_body,
        out_type=jax.ShapeDtypeStruct((E_PAD,), jnp.float32),
        mesh=_mesh,
        compiler_params=_sc_params,
        interpret=interpret,
        scratch_types=[
            pltpu.VMEM((N,), jnp.float32),
            pltpu.VMEM((SB2,), jnp.int32),
            pltpu.VMEM((SB2,), jnp.int32),
            pltpu.VMEM((SB2,), jnp.float32),
            pltpu.VMEM((SB2,), jnp.float32),
        ],
    )
    agg = pl.kernel(
        _sc_agg_body,
        out_type=jax.ShapeDtypeStruct((N, DP), jnp.float32),
        mesh=_mesh,
        compiler_params=_sc_params,
        interpret=interpret,
        scratch_types=[
            pltpu.VMEM((SB,), jnp.int32),
            pltpu.VMEM((SB,), jnp.int32),
            pltpu.VMEM((SB,), jnp.float32),
            pltpu.VMEM((2, CH, DP), jnp.float32),
            pltpu.VMEM((CH, DP), jnp.float32),
            pltpu.VMEM((CH,), jnp.int32),
            pltpu.VMEM_SHARED((SP_ROWS, DP), jnp.float32),
            pltpu.SemaphoreType.DMA,
        ],
    )
    return deg, nrm, agg


_sc_deg, _sc_norm, _sc_agg = _make_sc_kernels()


def _tc_dis_body(degp_ref, dis_ref):
    deg = jnp.sum(degp_ref[...], axis=0)
    pos = deg > 0
    dis_ref[...] = jnp.where(pos, lax.rsqrt(jnp.where(pos, deg, 1.0)), 0.0)


_tc_dis = pl.pallas_call(
    _tc_dis_body,
    out_shape=jax.ShapeDtypeStruct((N,), jnp.float32),
)


def _tc_stats_body(h_ref, s1_ref, s2_ref):
    @pl.when(pl.program_id(0) == 0)
    def _init():
        s1_ref[...] = jnp.zeros((1, D), jnp.float32)
        s2_ref[...] = jnp.zeros((1, D), jnp.float32)

    h = h_ref[...]
    s1_ref[...] += jnp.sum(h, axis=0, keepdims=True)
    s2_ref[...] += jnp.sum(h * h, axis=0, keepdims=True)


_tc_stats = pl.pallas_call(
    _tc_stats_body,
    grid=(N // BLK,),
    in_specs=[pl.BlockSpec((BLK, D), lambda i: (i, 0))],
    out_specs=(pl.BlockSpec((1, D), lambda i: (0, 0)),
               pl.BlockSpec((1, D), lambda i: (0, 0))),
    out_shape=(jax.ShapeDtypeStruct((1, D), jnp.float32),
               jax.ShapeDtypeStruct((1, D), jnp.float32)),
)


def _tc0_body(h_ref, s1_ref, s2_ref, gamma_ref, beta_ref, w1_ref, y_ref):
    mu = s1_ref[...] * (1.0 / N)
    var = s2_ref[...] * (1.0 / N) - mu * mu
    x = (gamma_ref[...] * (h_ref[...] - mu) / jnp.sqrt(var + EPS)
         + beta_ref[...])
    z = jnp.dot(x, w1_ref[...], preferred_element_type=jnp.float32)
    y_ref[...] = jnp.concatenate(
        [z, jnp.zeros((BLK, DP - D), jnp.float32)], axis=1)


_tc0 = pl.pallas_call(
    _tc0_body,
    grid=(N // BLK,),
    in_specs=[
        pl.BlockSpec((BLK, D), lambda i: (i, 0)),
        pl.BlockSpec((1, D), lambda i: (0, 0)),
        pl.BlockSpec((1, D), lambda i: (0, 0)),
        pl.BlockSpec((D,), lambda i: (0,)),
        pl.BlockSpec((D,), lambda i: (0,)),
        pl.BlockSpec((D, D), lambda i: (0, 0)),
    ],
    out_specs=pl.BlockSpec((BLK, DP), lambda i: (i, 0)),
    out_shape=jax.ShapeDtypeStruct((N, DP), jnp.float32),
)


def _tc_mid_body(agg_ref, b_ref, w_ref, y_ref):
    x = jax.nn.relu(agg_ref[...] + b_ref[...])
    z = jnp.dot(x, w_ref[...], preferred_element_type=jnp.float32)
    y_ref[...] = jnp.concatenate(
        [z, jnp.zeros((BLK, DP - D), jnp.float32)], axis=1)


_tc_mid = pl.pallas_call(
    _tc_mid_body,
    grid=(N // BLK,),
    in_specs=[
        pl.BlockSpec((BLK, DP), lambda i: (i, 0)),
        pl.BlockSpec((DP,), lambda i: (0,)),
        pl.BlockSpec((DP, D), lambda i: (0, 0)),
    ],
    out_specs=pl.BlockSpec((BLK, DP), lambda i: (i, 0)),
    out_shape=jax.ShapeDtypeStruct((N, DP), jnp.float32),
)


def _tc_fin_body(agg_ref, b3_ref, fw1_ref, fb1_ref, fw2_ref, fb2_ref,
                 fw3_ref, fb3_ref, fw4_ref, fb4_ref, out_ref):
    x = jax.nn.relu(agg_ref[...] + b3_ref[...])
    x = jax.nn.relu(
        jnp.dot(x, fw1_ref[...], preferred_element_type=jnp.float32)
        + fb1_ref[...])
    x = jax.nn.relu(
        jnp.dot(x, fw2_ref[...], preferred_element_type=jnp.float32)
        + fb2_ref[...])
    x = jax.nn.relu(
        jnp.dot(x, fw3_ref[...], preferred_element_type=jnp.float32)
        + fb3_ref[...])
    out_ref[...] = (
        jnp.dot(x, fw4_ref[...], preferred_element_type=jnp.float32)
        + fb4_ref[...])


_tc_fin = pl.pallas_call(
    _tc_fin_body,
    grid=(N // BLK,),
    in_specs=[
        pl.BlockSpec((BLK, DP), lambda i: (i, 0)),
        pl.BlockSpec((DP,), lambda i: (0,)),
        pl.BlockSpec((DP, D), lambda i: (0, 0)),
        pl.BlockSpec((D,), lambda i: (0,)),
        pl.BlockSpec((D, D), lambda i: (0, 0)),
        pl.BlockSpec((D,), lambda i: (0,)),
        pl.BlockSpec((D, D), lambda i: (0, 0)),
        pl.BlockSpec((D,), lambda i: (0,)),
        pl.BlockSpec((D, OUT), lambda i: (0, 0)),
        pl.BlockSpec((OUT,), lambda i: (0,)),
    ],
    out_specs=pl.BlockSpec((BLK, OUT), lambda i: (i, 0)),
    out_shape=jax.ShapeDtypeStruct((N, OUT), jnp.float32),
)


def kernel(h, edge_index, edge_weight, gamma, beta, W1, b1, W2, b2, W3, b3,
           fw1, fb1, fw2, fb2, fw3, fb3, fw4, fb4):
    row = edge_index[0]
    col = edge_index[1]
    padi = jnp.zeros((E_PAD - E,), jnp.int32)
    rowp = jnp.concatenate([row, padi])
    colp = jnp.concatenate([col, padi])
    ewp = jnp.concatenate([edge_weight, jnp.zeros((E_PAD - E,), jnp.float32)])

    padw = jnp.zeros((DP - D, D), jnp.float32)
    w2p = jnp.concatenate([W2, padw], axis=0)
    w3p = jnp.concatenate([W3, padw], axis=0)
    fw1p = jnp.concatenate([fw1, padw], axis=0)
    padb = jnp.zeros((DP - D,), jnp.float32)
    b1p = jnp.concatenate([b1, padb])
    b2p = jnp.concatenate([b2, padb])
    b3p = jnp.concatenate([b3, padb])

    degp = _sc_deg(colp, ewp)
    dis = _tc_dis(degp)
    s1, s2 = _tc_stats(h)
    y1 = _tc0(h, s1, s2, gamma, beta, W1)
    norm = _sc_norm(dis, rowp, colp, ewp)
    agg1 = _sc_agg(y1, rowp, colp, norm)
    y2 = _tc_mid(agg1, b1p, w2p)
    agg2 = _sc_agg(y2, rowp, colp, norm)
    y3 = _tc_mid(agg2, b2p, w3p)
    agg3 = _sc_agg(y3, rowp, colp, norm)
    return _tc_fin(agg3, b3p, fw1p, fb1, fw2, fb2, fw3, fb3, fw4, fb4)


# speedup vs baseline: 1.0103x; 1.0103x over previous
import jax
import jax.numpy as jnp
from jax import lax
from jax.experimental import pallas as pl
from jax.experimental.pallas import tpu as pltpu
from jax.experimental.pallas import tpu_sc as plsc

N = 50000
D = 66
DP = 72
OUT = 22
EPS = 1e-5

NC = 2
NS = 16
NW = NC * NS

E = 800000
CH = 64
SB = 1024
SB2 = 1024
NSB = 50
NSB2 = 25
EPT = NSB * SB
EPT2 = NSB2 * SB2
E_PAD = 16 * EPT

NHALF = N // 2
RPT = 1563
TRASH0 = 16 * RPT
SP_ROWS = TRASH0 + 256
BLK = 5000

_mesh = plsc.VectorSubcoreMesh(core_axis_name="c", subcore_axis_name="s")
_sc_params = pltpu.CompilerParams(
    needs_layout_passes=False, use_tc_tiling_on_sc=False)


def _sc_deg_body(col_hbm, ew_hbm, degp_hbm, colv, ewv, degv):
    cid = lax.axis_index("c")
    sid = lax.axis_index("s")
    wid = cid * NS + sid

    def _zero(i, _):
        degv[pl.ds(i * 16, 16)] = jnp.zeros((16,), jnp.float32)
        return 0

    lax.fori_loop(0, N // 16, _zero, 0)

    def _sb(s, _):
        eoff = wid * EPT2 + s * SB2
        pltpu.sync_copy(col_hbm.at[pl.ds(eoff, SB2)], colv)
        pltpu.sync_copy(ew_hbm.at[pl.ds(eoff, SB2)], ewv)

        def _q(q, _):
            c16 = colv[pl.ds(q * 16, 16)]
            w16 = ewv[pl.ds(q * 16, 16)]
            plsc.addupdate_scatter(degv, [c16], w16)
            return 0

        lax.fori_loop(0, SB2 // 16, _q, 0)
        return 0

    lax.fori_loop(0, NSB2, _sb, 0)
    pltpu.sync_copy(degv, degp_hbm.at[wid])


def _sc_norm_body(dis_hbm, row_hbm, col_hbm, ew_hbm, norm_hbm,
                  disv, rowv, colv, ewv, noutv):
    cid = lax.axis_index("c")
    sid = lax.axis_index("s")
    wid = cid * NS + sid
    pltpu.sync_copy(dis_hbm, disv)

    def _sb(s, _):
        eoff = wid * EPT2 + s * SB2
        pltpu.sync_copy(row_hbm.at[pl.ds(eoff, SB2)], rowv)
        pltpu.sync_copy(col_hbm.at[pl.ds(eoff, SB2)], colv)
        pltpu.sync_copy(ew_hbm.at[pl.ds(eoff, SB2)], ewv)

        def _q(q, _):
            r16 = rowv[pl.ds(q * 16, 16)]
            c16 = colv[pl.ds(q * 16, 16)]
            w16 = ewv[pl.ds(q * 16, 16)]
            dr = plsc.load_gather(disv, [r16])
            dc = plsc.load_gather(disv, [c16])
            noutv[pl.ds(q * 16, 16)] = dr * w16 * dc
            return 0

        lax.fori_loop(0, SB2 // 16, _q, 0)
        pltpu.sync_copy(noutv, norm_hbm.at[pl.ds(eoff, SB2)])
        return 0

    lax.fori_loop(0, NSB2, _sb, 0)


def _sc_agg_body(y_hbm, row_hbm, col_hbm, norm_hbm, agg_hbm,
                 rowi, coli, nrm, gath2, sgath, cidx, shared, sem):
    cid = lax.axis_index("c")
    sid = lax.axis_index("s")
    base = cid * NHALF
    NCH = SB // CH

    def _zb(i, _):
        for f in range(4):
            gath2[0, i, pl.ds(f * 16, 16)] = jnp.zeros((16,), jnp.float32)
        gath2[0, i, pl.ds(DP - 16, 16)] = jnp.zeros((16,), jnp.float32)
        return 0

    lax.fori_loop(0, CH, _zb, 0)

    def _zs(j, _):
        pltpu.sync_copy(gath2.at[0],
                        shared.at[pl.ds(sid * RPT + j * CH, CH)])
        return 0

    lax.fori_loop(0, RPT // CH, _zs, 0)
    pltpu.sync_copy(gath2.at[0, pl.ds(0, RPT % CH)],
                    shared.at[pl.ds(sid * RPT + RPT - RPT % CH, RPT % CH)])
    pltpu.sync_copy(gath2.at[0, pl.ds(0, 16)],
                    shared.at[pl.ds(TRASH0 + sid * 16, 16)])
    plsc.subcore_barrier()

    def _sb(s, _):
        eoff = sid * EPT + s * SB
        pltpu.sync_copy(row_hbm.at[pl.ds(eoff, SB)], rowi)
        pltpu.sync_copy(col_hbm.at[pl.ds(eoff, SB)], coli)
        pltpu.sync_copy(norm_hbm.at[pl.ds(eoff, SB)], nrm)
        pltpu.async_copy(y_hbm.at[rowi.at[pl.ds(0, CH)]], gath2.at[0], sem)

        def _pair(cp, _):
            for p in range(2):
                ch = cp * 2 + p
                co = ch * CH
                gbuf = gath2.at[p]
                pltpu.make_async_copy(
                    y_hbm.at[rowi.at[pl.ds(0, CH)]], gbuf, sem).wait()

                @pl.when(ch < NCH - 1)
                def _issue():
                    pltpu.async_copy(
                        y_hbm.at[rowi.at[pl.ds(co + CH, CH)]],
                        gath2.at[1 - p], sem)

                def _q(q, _):
                    cv = coli[pl.ds(co + q * 16, 16)] - base
                    okm = (cv >= 0) & (cv < NHALF)
                    tv = TRASH0 + sid * 16 + lax.iota(jnp.int32, 16)
                    cidx[pl.ds(q * 16, 16)] = jnp.where(okm, cv, tv)
                    wv = nrm[pl.ds(co + q * 16, 16)]
                    e0 = q * 16
                    for i in range(16):
                        w = wv[i]
                        for f in range(4):
                            sgath[e0 + i, pl.ds(f * 16, 16)] = (
                                gbuf[e0 + i, pl.ds(f * 16, 16)] * w)
                        sgath[e0 + i, pl.ds(DP - 16, 16)] = (
                            gbuf[e0 + i, pl.ds(DP - 16, 16)] * w)
                    return 0

                lax.fori_loop(0, CH // 16, _q, 0)
                pltpu.sync_copy(sgath, shared.at[cidx], add=True)
            return 0

        lax.fori_loop(0, NCH // 2, _pair, 0)
        return 0

    lax.fori_loop(0, NSB, _sb, 0)
    plsc.subcore_barrier()

    def _out(j, _):
        off = sid * RPT + j * CH
        pltpu.sync_copy(shared.at[pl.ds(off, CH)],
                        agg_hbm.at[pl.ds(cid * NHALF + off, CH)])
        return 0

    lax.fori_loop(0, RPT // CH, _out, 0)
    off2 = sid * RPT + RPT - RPT % CH

    @pl.when(sid < NS - 1)
    def _tail_full():
        pltpu.sync_copy(shared.at[pl.ds(off2, RPT % CH)],
                        agg_hbm.at[pl.ds(cid * NHALF + off2, RPT % CH)])

    @pl.when(sid == NS - 1)
    def _tail_last():
        rem = NHALF - (NS - 1) * RPT - (RPT // CH) * CH
        pltpu.sync_copy(shared.at[pl.ds(off2, rem)],
                        agg_hbm.at[pl.ds(cid * NHALF + off2, rem)])


def _make_sc_kernels(interpret=False):
    deg = pl.kernel(
        _sc_deg_body,
        out_type=jax.ShapeDtypeStruct((NW, N), jnp.float32),
        mesh=_mesh,
        compiler_params=_sc_params,
        interpret=interpret,
        scratch_types=[
            pltpu.VMEM((SB2,), jnp.int32),
            pltpu.VMEM((SB2,), jnp.float32),
            pltpu.VMEM((N,), jnp.float32),
        ],
    )
    nrm = pl.kernel(
        _sc_norm_body,
        out_type=jax.ShapeDtypeStruct((E_PAD,), jnp.float32),
        mesh=_mesh,
        compiler_params=_sc_params,
        interpret=interpret,
        scratch_types=[
            pltpu.VMEM((N,), jnp.float32),
            pltpu.VMEM((SB2,), jnp.int32),
            pltpu.VMEM((SB2,), jnp.int32),
            pltpu.VMEM((SB2,), jnp.float32),
            pltpu.VMEM((SB2,), jnp.float32),
        ],
    )
    agg = pl.kernel(
        _sc_agg_body,
        out_type=jax.ShapeDtypeStruct((N, DP), jnp.float32),
        mesh=_mesh,
        compiler_params=_sc_params,
        interpret=interpret,
        scratch_types=[
            pltpu.VMEM((SB,), jnp.int32),
            pltpu.VMEM((SB,), jnp.int32),
            pltpu.VMEM((SB,), jnp.float32),
            pltpu.VMEM((2, CH, DP), jnp.float32),
            pltpu.VMEM((CH, DP), jnp.float32),
            pltpu.VMEM((CH,), jnp.int32),
            pltpu.VMEM_SHARED((SP_ROWS, DP), jnp.float32),
            pltpu.SemaphoreType.DMA,
        ],
    )
    return deg, nrm, agg


_sc_deg, _sc_norm, _sc_agg = _make_sc_kernels()


def _tc_dis_body(degp_ref, dis_ref):
    deg = jnp.sum(degp_ref[...], axis=0)
    pos = deg > 0
    dis_ref[...] = jnp.where(pos, lax.rsqrt(jnp.where(pos, deg, 1.0)), 0.0)


_tc_dis = pl.pallas_call(
    _tc_dis_body,
    out_shape=jax.ShapeDtypeStruct((N,), jnp.float32),
)


def _tc_stats_body(h_ref, s1_ref, s2_ref):
    @pl.when(pl.program_id(0) == 0)
    def _init():
        s1_ref[...] = jnp.zeros((1, D), jnp.float32)
        s2_ref[...] = jnp.zeros((1, D), jnp.float32)

    h = h_ref[...]
    s1_ref[...] += jnp.sum(h, axis=0, keepdims=True)
    s2_ref[...] += jnp.sum(h * h, axis=0, keepdims=True)


_tc_stats = pl.pallas_call(
    _tc_stats_body,
    grid=(N // BLK,),
    in_specs=[pl.BlockSpec((BLK, D), lambda i: (i, 0))],
    out_specs=(pl.BlockSpec((1, D), lambda i: (0, 0)),
               pl.BlockSpec((1, D), lambda i: (0, 0))),
    out_shape=(jax.ShapeDtypeStruct((1, D), jnp.float32),
               jax.ShapeDtypeStruct((1, D), jnp.float32)),
)


def _tc0_body(h_ref, s1_ref, s2_ref, gamma_ref, beta_ref, w1_ref, y_ref):
    mu = s1_ref[...] * (1.0 / N)
    var = s2_ref[...] * (1.0 / N) - mu * mu
    x = (gamma_ref[...] * (h_ref[...] - mu) / jnp.sqrt(var + EPS)
         + beta_ref[...])
    z = jnp.dot(x, w1_ref[...], preferred_element_type=jnp.float32)
    y_ref[...] = jnp.concatenate(
        [z, jnp.zeros((BLK, DP - D), jnp.float32)], axis=1)


_tc0 = pl.pallas_call(
    _tc0_body,
    grid=(N // BLK,),
    in_specs=[
        pl.BlockSpec((BLK, D), lambda i: (i, 0)),
        pl.BlockSpec((1, D), lambda i: (0, 0)),
        pl.BlockSpec((1, D), lambda i: (0, 0)),
        pl.BlockSpec((D,), lambda i: (0,)),
        pl.BlockSpec((D,), lambda i: (0,)),
        pl.BlockSpec((D, D), lambda i: (0, 0)),
    ],
    out_specs=pl.BlockSpec((BLK, DP), lambda i: (i, 0)),
    out_shape=jax.ShapeDtypeStruct((N, DP), jnp.float32),
)


def _tc_mid_body(agg_ref, b_ref, w_ref, y_ref):
    x = jax.nn.relu(agg_ref[...] + b_ref[...])
    z = jnp.dot(x, w_ref[...], preferred_element_type=jnp.float32)
    y_ref[...] = jnp.concatenate(
        [z, jnp.zeros((BLK, DP - D), jnp.float32)], axis=1)


_tc_mid = pl.pallas_call(
    _tc_mid_body,
    grid=(N // BLK,),
    in_specs=[
        pl.BlockSpec((BLK, DP), lambda i: (i, 0)),
        pl.BlockSpec((DP,), lambda i: (0,)),
        pl.BlockSpec((DP, D), lambda i: (0, 0)),
    ],
    out_specs=pl.BlockSpec((BLK, DP), lambda i: (i, 0)),
    out_shape=jax.ShapeDtypeStruct((N, DP), jnp.float32),
)


def _tc_fin_body(agg_ref, b3_ref, fw1_ref, fb1_ref, fw2_ref, fb2_ref,
                 fw3_ref, fb3_ref, fw4_ref, fb4_ref, out_ref):
    x = jax.nn.relu(agg_ref[...] + b3_ref[...])
    x = jax.nn.relu(
        jnp.dot(x, fw1_ref[...], preferred_element_type=jnp.float32)
        + fb1_ref[...])
    x = jax.nn.relu(
        jnp.dot(x, fw2_ref[...], preferred_element_type=jnp.float32)
        + fb2_ref[...])
    x = jax.nn.relu(
        jnp.dot(x, fw3_ref[...], preferred_element_type=jnp.float32)
        + fb3_ref[...])
    out_ref[...] = (
        jnp.dot(x, fw4_ref[...], preferred_element_type=jnp.float32)
        + fb4_ref[...])


_tc_fin = pl.pallas_call(
    _tc_fin_body,
    grid=(N // BLK,),
    in_specs=[
        pl.BlockSpec((BLK, DP), lambda i: (i, 0)),
        pl.BlockSpec((DP,), lambda i: (0,)),
        pl.BlockSpec((DP, D), lambda i: (0, 0)),
        pl.BlockSpec((D,), lambda i: (0,)),
        pl.BlockSpec((D, D), lambda i: (0, 0)),
        pl.BlockSpec((D,), lambda i: (0,)),
        pl.BlockSpec((D, D), lambda i: (0, 0)),
        pl.BlockSpec((D,), lambda i: (0,)),
        pl.BlockSpec((D, OUT), lambda i: (0, 0)),
        pl.BlockSpec((OUT,), lambda i: (0,)),
    ],
    out_specs=pl.BlockSpec((BLK, OUT), lambda i: (i, 0)),
    out_shape=jax.ShapeDtypeStruct((N, OUT), jnp.float32),
)


def kernel(h, edge_index, edge_weight, gamma, beta, W1, b1, W2, b2, W3, b3,
           fw1, fb1, fw2, fb2, fw3, fb3, fw4, fb4):
    row = edge_index[0]
    col = edge_index[1]
    padi = jnp.zeros((E_PAD - E,), jnp.int32)
    rowp = jnp.concatenate([row, padi])
    colp = jnp.concatenate([col, padi])
    ewp = jnp.concatenate([edge_weight, jnp.zeros((E_PAD - E,), jnp.float32)])

    padw = jnp.zeros((DP - D, D), jnp.float32)
    w2p = jnp.concatenate([W2, padw], axis=0)
    w3p = jnp.concatenate([W3, padw], axis=0)
    fw1p = jnp.concatenate([fw1, padw], axis=0)
    padb = jnp.zeros((DP - D,), jnp.float32)
    b1p = jnp.concatenate([b1, padb])
    b2p = jnp.concatenate([b2, padb])
    b3p = jnp.concatenate([b3, padb])

    degp = _sc_deg(colp, ewp)
    dis = _tc_dis(degp)
    s1, s2 = _tc_stats(h)
    y1 = _tc0(h, s1, s2, gamma, beta, W1)
    norm = _sc_norm(dis, rowp, colp, ewp)
    agg1 = _sc_agg(y1, rowp, colp, norm)
    y2 = _tc_mid(agg1, b1p, w2p)
    agg2 = _sc_agg(y2, rowp, colp, norm)
    y3 = _tc_mid(agg2, b2p, w3p)
    agg3 = _sc_agg(y3, rowp, colp, norm)
    return _tc_fin(agg3, b3p, fw1p, fb1, fw2, fb2, fw3, fb3, fw4, fb4)
